# Initial kernel scaffold; baseline (speedup 1.0000x reference)
import jax
import jax.numpy as jnp
from jax.experimental import pallas as pl


def kernel(x, edge_index, edge_type, edge_norm, Wrel1, Wself1, Wrel, Wself, Wc1, bc1, Wc2, bc2):
    raise NotImplementedError("write your pallas kernel here")


# speedup vs baseline: 12.0803x; 12.0803x over previous
import functools

import jax
import jax.numpy as jnp
from jax import lax
from jax.experimental import pallas as pl
from jax.experimental.pallas import tpu as pltpu
from jax.experimental.pallas import tpu_sc as plsc

N = 10000
E = 320000
G_DIM = 200
H_DIM = 80
NUM_LAYERS = 5
N_REL = 8
HC_DIM = 100
TAG_SIZE = 6

LANES = 128
EP = 327680
ROWS = EP // LANES
NW = 32
ROWS_PER_W = ROWS // NW
CHUNK_ROWS = 4
NCHUNK = ROWS_PER_W // CHUNK_ROWS
NB = 2000


def _transform_body(h_ref, w_ref, t_ref, s_ref):
    t = jnp.dot(h_ref[...], w_ref[...], preferred_element_type=jnp.float32)
    t_ref[...] = t[:, : N_REL * H_DIM]
    s_ref[...] = t[:, N_REL * H_DIM:]


def _transform(h, wcat):
    d = h.shape[1]
    return pl.pallas_call(
        _transform_body,
        grid=(N // NB,),
        in_specs=[
            pl.BlockSpec((NB, d), lambda i: (i, 0)),
            pl.BlockSpec((d, (N_REL + 1) * H_DIM), lambda i: (0, 0)),
        ],
        out_specs=[
            pl.BlockSpec((NB, N_REL * H_DIM), lambda i: (i, 0)),
            pl.BlockSpec((NB, H_DIM), lambda i: (i, 0)),
        ],
        out_shape=[
            jax.ShapeDtypeStruct((N, N_REL * H_DIM), jnp.float32),
            jax.ShapeDtypeStruct((N, H_DIM), jnp.float32),
        ],
    )(h, wcat)


def _combine_body(p_ref, s_ref, o_ref):
    o_ref[...] = jnp.maximum(p_ref[0] + p_ref[1] + s_ref[...], 0.0)


def _combine(partials, selfh):
    return pl.pallas_call(
        _combine_body,
        grid=(N // NB,),
        in_specs=[
            pl.BlockSpec((2, NB, H_DIM), lambda i: (0, i, 0)),
            pl.BlockSpec((NB, H_DIM), lambda i: (i, 0)),
        ],
        out_specs=pl.BlockSpec((NB, H_DIM), lambda i: (i, 0)),
        out_shape=jax.ShapeDtypeStruct((N, H_DIM), jnp.float32),
    )(partials, selfh)


def _classifier_body(x_ref, h_ref, w1a_ref, w1b_ref, b1_ref, w2_ref, b2_ref, o_ref):
    hid = jnp.dot(x_ref[...], w1a_ref[...], preferred_element_type=jnp.float32)
    hid = hid + jnp.dot(h_ref[...], w1b_ref[...], preferred_element_type=jnp.float32)
    hid = jnp.maximum(hid + b1_ref[...], 0.0)
    o_ref[...] = jnp.dot(hid, w2_ref[...], preferred_element_type=jnp.float32) + b2_ref[...]


def _classifier(x, h, w1a, w1b, b1, w2, b2):
    return pl.pallas_call(
        _classifier_body,
        grid=(N // NB,),
        in_specs=[
            pl.BlockSpec((NB, G_DIM), lambda i: (i, 0)),
            pl.BlockSpec((NB, H_DIM), lambda i: (i, 0)),
            pl.BlockSpec((G_DIM, HC_DIM), lambda i: (0, 0)),
            pl.BlockSpec((H_DIM, HC_DIM), lambda i: (0, 0)),
            pl.BlockSpec((1, HC_DIM), lambda i: (0, 0)),
            pl.BlockSpec((HC_DIM, TAG_SIZE), lambda i: (0, 0)),
            pl.BlockSpec((1, TAG_SIZE), lambda i: (0, 0)),
        ],
        out_specs=pl.BlockSpec((NB, TAG_SIZE), lambda i: (i, 0)),
        out_shape=jax.ShapeDtypeStruct((N, TAG_SIZE), jnp.float32),
    )(x, h, w1a, w1b, b1, w2, b2)


def _sc_body(table, src_h, et_h, en_h, dst_h, zeros_h, out,
             src_v, et_v, en_v, dst_v, gidx_v, rows_v, acc, sem):
    core = lax.axis_index("c")
    sid = lax.axis_index("s")
    wid = sid * 2 + core

    pltpu.sync_copy(zeros_h.at[pl.ds(sid * 624, 624)],
                    acc.at[pl.ds(sid * 624, 624)])

    @pl.when(sid == 15)
    def _():
        pltpu.sync_copy(zeros_h.at[pl.ds(9984, 16)], acc.at[pl.ds(9984, 16)])

    plsc.subcore_barrier()

    row_base = wid * ROWS_PER_W

    @pl.loop(0, NCHUNK)
    def _chunks(ch):
        rb = row_base + ch * CHUNK_ROWS
        pltpu.sync_copy(src_h.at[pl.ds(rb, CHUNK_ROWS)], src_v)
        pltpu.sync_copy(et_h.at[pl.ds(rb, CHUNK_ROWS)], et_v)
        pltpu.sync_copy(en_h.at[pl.ds(rb, CHUNK_ROWS)], en_v)
        pltpu.sync_copy(dst_h.at[pl.ds(rb, CHUNK_ROWS)], dst_v)

        for r in range(CHUNK_ROWS):
            for k in range(LANES // 16):
                sl = pl.ds(k * 16, 16)
                gidx_v[r, sl] = src_v[r, sl] * N_REL + et_v[r, sl]

        copies = []
        for r in range(CHUNK_ROWS):
            copies.append(
                pltpu.async_copy(table.at[gidx_v.at[r]], rows_v.at[r], sem))
        for c in copies:
            c.wait()

        for r in range(CHUNK_ROWS):
            @pl.loop(0, LANES // 16)
            def _scale(g):
                env = en_v[r, pl.ds(g * 16, 16)]
                for j in range(16):
                    e = env[j]
                    i = g * 16 + j
                    for k in range(H_DIM // 16):
                        sl = pl.ds(k * 16, 16)
                        rows_v[r, i, sl] = rows_v[r, i, sl] * e

        for r in range(CHUNK_ROWS):
            pltpu.sync_copy(rows_v.at[r], acc.at[dst_v.at[r]], add=True)

    plsc.subcore_barrier()

    pltpu.sync_copy(acc.at[pl.ds(sid * 624, 624)],
                    out.at[core, pl.ds(sid * 624, 624)])

    @pl.when(sid == 15)
    def _():
        pltpu.sync_copy(acc.at[pl.ds(9984, 16)], out.at[core, pl.ds(9984, 16)])


def _sc_aggregate(table, src2, et2, en2, dst2, zeros):
    k = pl.kernel(
        _sc_body,
        out_type=jax.ShapeDtypeStruct((2, N, H_DIM), jnp.float32),
        mesh=plsc.VectorSubcoreMesh(core_axis_name="c", subcore_axis_name="s"),
        compiler_params=pltpu.CompilerParams(use_tc_tiling_on_sc=False),
        scratch_types=[
            pltpu.VMEM((CHUNK_ROWS, LANES), jnp.int32),
            pltpu.VMEM((CHUNK_ROWS, LANES), jnp.int32),
            pltpu.VMEM((CHUNK_ROWS, LANES), jnp.float32),
            pltpu.VMEM((CHUNK_ROWS, LANES), jnp.int32),
            pltpu.VMEM((CHUNK_ROWS, LANES), jnp.int32),
            pltpu.VMEM((CHUNK_ROWS, LANES, H_DIM), jnp.float32),
            pltpu.VMEM_SHARED((N, H_DIM), jnp.float32),
            pltpu.SemaphoreType.DMA,
        ],
    )
    return k(table, src2, et2, en2, dst2, zeros)


def kernel(x, edge_index, edge_type, edge_norm, Wrel1, Wself1, Wrel, Wself,
           Wc1, bc1, Wc2, bc2):
    wcat1 = jnp.concatenate(
        [Wrel1.transpose(1, 0, 2).reshape(G_DIM, N_REL * H_DIM), Wself1], axis=1)
    wcats = [wcat1]
    for i in range(NUM_LAYERS - 1):
        wcats.append(jnp.concatenate(
            [Wrel[i].transpose(1, 0, 2).reshape(H_DIM, N_REL * H_DIM), Wself[i]],
            axis=1))

    pad = EP - E
    src2 = jnp.concatenate(
        [edge_index[0], jnp.zeros((pad,), jnp.int32)]).reshape(ROWS, LANES)
    dst2 = jnp.concatenate(
        [edge_index[1], jnp.zeros((pad,), jnp.int32)]).reshape(ROWS, LANES)
    et2 = jnp.concatenate(
        [edge_type, jnp.zeros((pad,), jnp.int32)]).reshape(ROWS, LANES)
    en2 = jnp.concatenate(
        [edge_norm, jnp.zeros((pad,), jnp.float32)]).reshape(ROWS, LANES)
    zeros = jnp.zeros((N, H_DIM), jnp.float32)

    h = x
    for l in range(NUM_LAYERS):
        t_rel, selfh = _transform(h, wcats[l])
        table = t_rel.reshape(N * N_REL, H_DIM)
        partials = _sc_aggregate(table, src2, et2, en2, dst2, zeros)
        h = _combine(partials, selfh)

    logits = _classifier(x, h, Wc1[:G_DIM], Wc1[G_DIM:],
                         bc1.reshape(1, HC_DIM), Wc2, bc2.reshape(1, TAG_SIZE))
    return logits


# speedup vs baseline: 14.9018x; 1.2336x over previous
import functools

import jax
import jax.numpy as jnp
from jax import lax
from jax.experimental import pallas as pl
from jax.experimental.pallas import tpu as pltpu
from jax.experimental.pallas import tpu_sc as plsc

N = 10000
E = 320000
G_DIM = 200
H_DIM = 80
NUM_LAYERS = 5
N_REL = 8
HC_DIM = 100
TAG_SIZE = 6

LANES = 128
EP = 327680
ROWS = EP // LANES
NW = 32
ROWS_PER_W = ROWS // NW
CHUNK_ROWS = 1
NCHUNK = ROWS_PER_W // CHUNK_ROWS
NBUF = 4
NB = 2000


def _transform_body(h_ref, w_ref, t_ref, s_ref):
    t = jnp.dot(h_ref[...], w_ref[...], preferred_element_type=jnp.float32)
    t_ref[...] = t[:, : N_REL * H_DIM]
    s_ref[...] = t[:, N_REL * H_DIM:]


def _transform(h, wcat):
    d = h.shape[1]
    return pl.pallas_call(
        _transform_body,
        grid=(N // NB,),
        in_specs=[
            pl.BlockSpec((NB, d), lambda i: (i, 0)),
            pl.BlockSpec((d, (N_REL + 1) * H_DIM), lambda i: (0, 0)),
        ],
        out_specs=[
            pl.BlockSpec((NB, N_REL * H_DIM), lambda i: (i, 0)),
            pl.BlockSpec((NB, H_DIM), lambda i: (i, 0)),
        ],
        out_shape=[
            jax.ShapeDtypeStruct((N, N_REL * H_DIM), jnp.float32),
            jax.ShapeDtypeStruct((N, H_DIM), jnp.float32),
        ],
    )(h, wcat)


def _combine_body(p_ref, s_ref, o_ref):
    o_ref[...] = jnp.maximum(p_ref[0] + p_ref[1] + s_ref[...], 0.0)


def _combine(partials, selfh):
    return pl.pallas_call(
        _combine_body,
        grid=(N // NB,),
        in_specs=[
            pl.BlockSpec((2, NB, H_DIM), lambda i: (0, i, 0)),
            pl.BlockSpec((NB, H_DIM), lambda i: (i, 0)),
        ],
        out_specs=pl.BlockSpec((NB, H_DIM), lambda i: (i, 0)),
        out_shape=jax.ShapeDtypeStruct((N, H_DIM), jnp.float32),
    )(partials, selfh)


def _classifier_body(x_ref, h_ref, w1a_ref, w1b_ref, b1_ref, w2_ref, b2_ref, o_ref):
    hid = jnp.dot(x_ref[...], w1a_ref[...], preferred_element_type=jnp.float32)
    hid = hid + jnp.dot(h_ref[...], w1b_ref[...], preferred_element_type=jnp.float32)
    hid = jnp.maximum(hid + b1_ref[...], 0.0)
    o_ref[...] = jnp.dot(hid, w2_ref[...], preferred_element_type=jnp.float32) + b2_ref[...]


def _classifier(x, h, w1a, w1b, b1, w2, b2):
    return pl.pallas_call(
        _classifier_body,
        grid=(N // NB,),
        in_specs=[
            pl.BlockSpec((NB, G_DIM), lambda i: (i, 0)),
            pl.BlockSpec((NB, H_DIM), lambda i: (i, 0)),
            pl.BlockSpec((G_DIM, HC_DIM), lambda i: (0, 0)),
            pl.BlockSpec((H_DIM, HC_DIM), lambda i: (0, 0)),
            pl.BlockSpec((1, HC_DIM), lambda i: (0, 0)),
            pl.BlockSpec((HC_DIM, TAG_SIZE), lambda i: (0, 0)),
            pl.BlockSpec((1, TAG_SIZE), lambda i: (0, 0)),
        ],
        out_specs=pl.BlockSpec((NB, TAG_SIZE), lambda i: (i, 0)),
        out_shape=jax.ShapeDtypeStruct((N, TAG_SIZE), jnp.float32),
    )(x, h, w1a, w1b, b1, w2, b2)


def _sc_body(table, pkt_h, en_h, zeros_h, out,
             pkt_v, en_v, gidx_v, rows_v, acc,
             sem_g, sem_s):
    core = lax.axis_index("c")
    sid = lax.axis_index("s")
    wid = sid * 2 + core
    row_base = wid * ROWS_PER_W

    def stage(ch, b):
        rb = row_base + ch * CHUNK_ROWS
        pltpu.sync_copy(pkt_h.at[pl.ds(rb, CHUNK_ROWS)], pkt_v.at[b])
        pltpu.sync_copy(en_h.at[pl.ds(rb, CHUNK_ROWS)], en_v.at[b])
        for r in range(CHUNK_ROWS):
            for k in range(LANES // 16):
                sl = pl.ds(k * 16, 16)
                gidx_v[b, r, sl] = (pkt_v[b, r, 0, sl] * N_REL
                                    + pkt_v[b, r, 1, sl])
        for r in range(CHUNK_ROWS):
            pltpu.async_copy(table.at[gidx_v.at[b, r]], rows_v.at[b, r],
                             sem_g[b])

    def wait_gather(b):
        for r in range(CHUNK_ROWS):
            pltpu.make_async_copy(table.at[gidx_v.at[b, r]],
                                  rows_v.at[b, r], sem_g[b]).wait()

    def drain_scatter(b):
        for r in range(CHUNK_ROWS):
            pltpu.make_async_copy(rows_v.at[b, r],
                                  acc.at[pkt_v.at[b, r, 2]], sem_s[b]).wait()

    def process(b):
        wait_gather(b)
        for r in range(CHUNK_ROWS):
            @pl.loop(0, LANES // 16)
            def _scale(g):
                env = en_v[b, r, pl.ds(g * 16, 16)]
                for j in range(16):
                    e = env[j]
                    i = g * 16 + j
                    for k in range(H_DIM // 16):
                        sl = pl.ds(k * 16, 16)
                        rows_v[b, r, i, sl] = rows_v[b, r, i, sl] * e
        for r in range(CHUNK_ROWS):
            pltpu.async_copy(rows_v.at[b, r], acc.at[pkt_v.at[b, r, 2]],
                             sem_s[b], add=True)

    pltpu.sync_copy(zeros_h.at[pl.ds(sid * 624, 624)],
                    acc.at[pl.ds(sid * 624, 624)])

    @pl.when(sid == 15)
    def _():
        pltpu.sync_copy(zeros_h.at[pl.ds(9984, 16)], acc.at[pl.ds(9984, 16)])

    plsc.subcore_barrier()

    stage(0, 0)

    @pl.loop(0, NCHUNK // NBUF)
    def _chunks(i):
        for b in range(NBUF):
            ch = i * NBUF + b
            nb = (b + 1) % NBUF
            if b < NBUF - 1:
                @pl.when(i > 0)
                def _():
                    drain_scatter(nb)
                stage(ch + 1, nb)
            else:
                drain_scatter(nb)

                @pl.when(i < NCHUNK // NBUF - 1)
                def _():
                    stage(ch + 1, nb)
            process(b)

    for b in range(1, NBUF):
        drain_scatter(b)

    plsc.subcore_barrier()

    pltpu.sync_copy(acc.at[pl.ds(sid * 624, 624)],
                    out.at[core, pl.ds(sid * 624, 624)])

    @pl.when(sid == 15)
    def _():
        pltpu.sync_copy(acc.at[pl.ds(9984, 16)], out.at[core, pl.ds(9984, 16)])


def _sc_aggregate(table, pkt, en2, zeros):
    k = pl.kernel(
        _sc_body,
        out_type=jax.ShapeDtypeStruct((2, N, H_DIM), jnp.float32),
        mesh=plsc.VectorSubcoreMesh(core_axis_name="c", subcore_axis_name="s"),
        compiler_params=pltpu.CompilerParams(use_tc_tiling_on_sc=False),
        scratch_types=[
            pltpu.VMEM((NBUF, CHUNK_ROWS, 3, LANES), jnp.int32),
            pltpu.VMEM((NBUF, CHUNK_ROWS, LANES), jnp.float32),
            pltpu.VMEM((NBUF, CHUNK_ROWS, LANES), jnp.int32),
            pltpu.VMEM((NBUF, CHUNK_ROWS, LANES, H_DIM), jnp.float32),
            pltpu.VMEM_SHARED((N, H_DIM), jnp.float32),
            [pltpu.SemaphoreType.DMA] * NBUF,
            [pltpu.SemaphoreType.DMA] * NBUF,
        ],
    )
    return k(table, pkt, en2, zeros)


def kernel(x, edge_index, edge_type, edge_norm, Wrel1, Wself1, Wrel, Wself,
           Wc1, bc1, Wc2, bc2):
    wcat1 = jnp.concatenate(
        [Wrel1.transpose(1, 0, 2).reshape(G_DIM, N_REL * H_DIM), Wself1], axis=1)
    wcats = [wcat1]
    for i in range(NUM_LAYERS - 1):
        wcats.append(jnp.concatenate(
            [Wrel[i].transpose(1, 0, 2).reshape(H_DIM, N_REL * H_DIM), Wself[i]],
            axis=1))

    pad = EP - E

    def _padded(a, dtype):
        return jnp.concatenate(
            [a.astype(dtype), jnp.zeros((pad,), dtype)]).reshape(ROWS, LANES)

    pkt = jnp.stack([
        _padded(edge_index[0], jnp.int32),
        _padded(edge_type, jnp.int32),
        _padded(edge_index[1], jnp.int32),
    ], axis=1)
    en2 = _padded(edge_norm, jnp.float32)
    zeros = jnp.zeros((N, H_DIM), jnp.float32)

    h = x
    for l in range(NUM_LAYERS):
        t_rel, selfh = _transform(h, wcats[l])
        table = t_rel.reshape(N * N_REL, H_DIM)
        partials = _sc_aggregate(table, pkt, en2, zeros)
        h = _combine(partials, selfh)

    logits = _classifier(x, h, Wc1[:G_DIM], Wc1[G_DIM:],
                         bc1.reshape(1, HC_DIM), Wc2, bc2.reshape(1, TAG_SIZE))
    return logits


# speedup vs baseline: 15.9296x; 1.0690x over previous
import functools

import jax
import jax.numpy as jnp
from jax import lax
from jax.experimental import pallas as pl
from jax.experimental.pallas import tpu as pltpu
from jax.experimental.pallas import tpu_sc as plsc

N = 10000
E = 320000
G_DIM = 200
H_DIM = 80
NUM_LAYERS = 5
N_REL = 8
HC_DIM = 100
TAG_SIZE = 6

LANES = 128
EP = 327680
ROWS = EP // LANES
NW = 32
ROWS_PER_W = ROWS // NW
CHUNK_ROWS = 1
NCHUNK = ROWS_PER_W // CHUNK_ROWS
NBUF = 4
NB = 2000


def _transform_body(h_ref, w_ref, t_ref, s_ref):
    t = jnp.dot(h_ref[...], w_ref[...], preferred_element_type=jnp.float32)
    t_ref[...] = t[:, : N_REL * H_DIM]
    s_ref[...] = t[:, N_REL * H_DIM:]


def _transform(h, wcat):
    d = h.shape[1]
    return pl.pallas_call(
        _transform_body,
        grid=(N // NB,),
        in_specs=[
            pl.BlockSpec((NB, d), lambda i: (i, 0)),
            pl.BlockSpec((d, (N_REL + 1) * H_DIM), lambda i: (0, 0)),
        ],
        out_specs=[
            pl.BlockSpec((NB, N_REL * H_DIM), lambda i: (i, 0)),
            pl.BlockSpec((NB, H_DIM), lambda i: (i, 0)),
        ],
        out_shape=[
            jax.ShapeDtypeStruct((N, N_REL * H_DIM), jnp.float32),
            jax.ShapeDtypeStruct((N, H_DIM), jnp.float32),
        ],
    )(h, wcat)


def _transform_fused_body(p_ref, s_ref, w_ref, t_ref, s2_ref):
    h = jnp.maximum(p_ref[0] + p_ref[1] + s_ref[...], 0.0)
    t = jnp.dot(h, w_ref[...], preferred_element_type=jnp.float32)
    t_ref[...] = t[:, : N_REL * H_DIM]
    s2_ref[...] = t[:, N_REL * H_DIM:]


def _transform_fused(partials, selfh, wcat):
    return pl.pallas_call(
        _transform_fused_body,
        grid=(N // NB,),
        in_specs=[
            pl.BlockSpec((2, NB, H_DIM), lambda i: (0, i, 0)),
            pl.BlockSpec((NB, H_DIM), lambda i: (i, 0)),
            pl.BlockSpec((H_DIM, (N_REL + 1) * H_DIM), lambda i: (0, 0)),
        ],
        out_specs=[
            pl.BlockSpec((NB, N_REL * H_DIM), lambda i: (i, 0)),
            pl.BlockSpec((NB, H_DIM), lambda i: (i, 0)),
        ],
        out_shape=[
            jax.ShapeDtypeStruct((N, N_REL * H_DIM), jnp.float32),
            jax.ShapeDtypeStruct((N, H_DIM), jnp.float32),
        ],
    )(partials, selfh, wcat)


def _classifier_body(x_ref, p_ref, s_ref, w1a_ref, w1b_ref, b1_ref, w2_ref,
                     b2_ref, o_ref):
    h = jnp.maximum(p_ref[0] + p_ref[1] + s_ref[...], 0.0)
    hid = jnp.dot(x_ref[...], w1a_ref[...], preferred_element_type=jnp.float32)
    hid = hid + jnp.dot(h, w1b_ref[...], preferred_element_type=jnp.float32)
    hid = jnp.maximum(hid + b1_ref[...], 0.0)
    o_ref[...] = jnp.dot(hid, w2_ref[...], preferred_element_type=jnp.float32) + b2_ref[...]


def _classifier(x, partials, selfh, w1a, w1b, b1, w2, b2):
    return pl.pallas_call(
        _classifier_body,
        grid=(N // NB,),
        in_specs=[
            pl.BlockSpec((NB, G_DIM), lambda i: (i, 0)),
            pl.BlockSpec((2, NB, H_DIM), lambda i: (0, i, 0)),
            pl.BlockSpec((NB, H_DIM), lambda i: (i, 0)),
            pl.BlockSpec((G_DIM, HC_DIM), lambda i: (0, 0)),
            pl.BlockSpec((H_DIM, HC_DIM), lambda i: (0, 0)),
            pl.BlockSpec((1, HC_DIM), lambda i: (0, 0)),
            pl.BlockSpec((HC_DIM, TAG_SIZE), lambda i: (0, 0)),
            pl.BlockSpec((1, TAG_SIZE), lambda i: (0, 0)),
        ],
        out_specs=pl.BlockSpec((NB, TAG_SIZE), lambda i: (i, 0)),
        out_shape=jax.ShapeDtypeStruct((N, TAG_SIZE), jnp.float32),
    )(x, partials, selfh, w1a, w1b, b1, w2, b2)


def _sc_body(table, pkt_h, en_h, zeros_h, out,
             pkt_v, en_v, gidx_v, rows_v, acc,
             sem_i, sem_g, sem_s):
    core = lax.axis_index("c")
    sid = lax.axis_index("s")
    wid = sid * 2 + core
    row_base = wid * ROWS_PER_W
    niter = NCHUNK // NBUF

    def fire_idx(ch, b):
        rb = row_base + ch * CHUNK_ROWS
        pltpu.async_copy(pkt_h.at[pl.ds(rb, CHUNK_ROWS)], pkt_v.at[b],
                         sem_i[b])
        pltpu.async_copy(en_h.at[pl.ds(rb, CHUNK_ROWS)], en_v.at[b],
                         sem_i[b])

    def fire_gather(ch, b):
        rb = row_base + ch * CHUNK_ROWS
        pltpu.make_async_copy(pkt_h.at[pl.ds(rb, CHUNK_ROWS)], pkt_v.at[b],
                              sem_i[b]).wait()
        pltpu.make_async_copy(en_h.at[pl.ds(rb, CHUNK_ROWS)], en_v.at[b],
                              sem_i[b]).wait()
        for r in range(CHUNK_ROWS):
            for k in range(LANES // 16):
                sl = pl.ds(k * 16, 16)
                gidx_v[b, r, sl] = (pkt_v[b, r, 0, sl] * N_REL
                                    + pkt_v[b, r, 1, sl])
        for r in range(CHUNK_ROWS):
            pltpu.async_copy(table.at[gidx_v.at[b, r]], rows_v.at[b, r],
                             sem_g[b])

    def drain_scatter(b):
        for r in range(CHUNK_ROWS):
            pltpu.make_async_copy(rows_v.at[b, r],
                                  acc.at[pkt_v.at[b, r, 2]], sem_s[b]).wait()

    def process(b):
        for r in range(CHUNK_ROWS):
            pltpu.make_async_copy(table.at[gidx_v.at[b, r]],
                                  rows_v.at[b, r], sem_g[b]).wait()
        for r in range(CHUNK_ROWS):
            @pl.loop(0, LANES // 16)
            def _scale(g):
                env = en_v[b, r, pl.ds(g * 16, 16)]
                for j in range(16):
                    e = env[j]
                    i = g * 16 + j
                    for k in range(H_DIM // 16):
                        sl = pl.ds(k * 16, 16)
                        rows_v[b, r, i, sl] = rows_v[b, r, i, sl] * e
        for r in range(CHUNK_ROWS):
            pltpu.async_copy(rows_v.at[b, r], acc.at[pkt_v.at[b, r, 2]],
                             sem_s[b], add=True)

    pltpu.sync_copy(zeros_h.at[pl.ds(sid * 624, 624)],
                    acc.at[pl.ds(sid * 624, 624)])

    @pl.when(sid == 15)
    def _():
        pltpu.sync_copy(zeros_h.at[pl.ds(9984, 16)], acc.at[pl.ds(9984, 16)])

    plsc.subcore_barrier()

    fire_idx(0, 0)
    fire_idx(1, 1)
    fire_gather(0, 0)

    @pl.loop(0, niter)
    def _chunks(i):
        for b in range(NBUF):
            ch = i * NBUF + b
            b2 = (b + 2) % NBUF
            b1 = (b + 1) % NBUF
            if b >= 2:
                drain_scatter(b2)

                @pl.when(i < niter - 1)
                def _():
                    fire_idx(ch + 2, b2)
            else:
                @pl.when(i > 0)
                def _():
                    drain_scatter(b2)
                fire_idx(ch + 2, b2)
            if b < NBUF - 1:
                fire_gather(ch + 1, b1)
            else:
                @pl.when(i < niter - 1)
                def _():
                    fire_gather(ch + 1, b1)
            process(b)

    drain_scatter(NBUF - 2)
    drain_scatter(NBUF - 1)

    plsc.subcore_barrier()

    pltpu.sync_copy(acc.at[pl.ds(sid * 624, 624)],
                    out.at[core, pl.ds(sid * 624, 624)])

    @pl.when(sid == 15)
    def _():
        pltpu.sync_copy(acc.at[pl.ds(9984, 16)], out.at[core, pl.ds(9984, 16)])


def _sc_aggregate(table, pkt, en2, zeros):
    k = pl.kernel(
        _sc_body,
        out_type=jax.ShapeDtypeStruct((2, N, H_DIM), jnp.float32),
        mesh=plsc.VectorSubcoreMesh(core_axis_name="c", subcore_axis_name="s"),
        compiler_params=pltpu.CompilerParams(use_tc_tiling_on_sc=False),
        scratch_types=[
            pltpu.VMEM((NBUF, CHUNK_ROWS, 3, LANES), jnp.int32),
            pltpu.VMEM((NBUF, CHUNK_ROWS, LANES), jnp.float32),
            pltpu.VMEM((NBUF, CHUNK_ROWS, LANES), jnp.int32),
            pltpu.VMEM((NBUF, CHUNK_ROWS, LANES, H_DIM), jnp.float32),
            pltpu.VMEM_SHARED((N, H_DIM), jnp.float32),
            [pltpu.SemaphoreType.DMA] * NBUF,
            [pltpu.SemaphoreType.DMA] * NBUF,
            [pltpu.SemaphoreType.DMA] * NBUF,
        ],
    )
    return k(table, pkt, en2, zeros)


def kernel(x, edge_index, edge_type, edge_norm, Wrel1, Wself1, Wrel, Wself,
           Wc1, bc1, Wc2, bc2):
    wcat1 = jnp.concatenate(
        [Wrel1.transpose(1, 0, 2).reshape(G_DIM, N_REL * H_DIM), Wself1], axis=1)
    wcats = [wcat1]
    for i in range(NUM_LAYERS - 1):
        wcats.append(jnp.concatenate(
            [Wrel[i].transpose(1, 0, 2).reshape(H_DIM, N_REL * H_DIM), Wself[i]],
            axis=1))

    pad = EP - E

    def _padded(a, dtype):
        return jnp.concatenate(
            [a.astype(dtype), jnp.zeros((pad,), dtype)]).reshape(ROWS, LANES)

    pkt = jnp.stack([
        _padded(edge_index[0], jnp.int32),
        _padded(edge_type, jnp.int32),
        _padded(edge_index[1], jnp.int32),
    ], axis=1)
    en2 = _padded(edge_norm, jnp.float32)
    zeros = jnp.zeros((N, H_DIM), jnp.float32)

    t_rel, selfh = _transform(x, wcats[0])
    partials = _sc_aggregate(t_rel.reshape(N * N_REL, H_DIM), pkt, en2, zeros)
    for l in range(1, NUM_LAYERS):
        t_rel, selfh = _transform_fused(partials, selfh, wcats[l])
        partials = _sc_aggregate(t_rel.reshape(N * N_REL, H_DIM), pkt, en2, zeros)

    logits = _classifier(x, partials, selfh, Wc1[:G_DIM], Wc1[G_DIM:],
                         bc1.reshape(1, HC_DIM), Wc2, bc2.reshape(1, TAG_SIZE))
    return logits


# speedup vs baseline: 44.7683x; 2.8104x over previous
import functools

import jax
import jax.numpy as jnp
from jax import lax
from jax.experimental import pallas as pl
from jax.experimental.pallas import tpu as pltpu
from jax.experimental.pallas import tpu_sc as plsc

N = 10000
E = 320000
G_DIM = 200
H_DIM = 80
NUM_LAYERS = 5
N_REL = 8
HC_DIM = 100
TAG_SIZE = 6

LANES = 128
EP = 327680
ROWS = EP // LANES
NW = 32
ROWS_PER_W = ROWS // NW
CHUNK_ROWS = 1
NCHUNK = ROWS_PER_W // CHUNK_ROWS
NBUF = 5
NSPLIT = 2
NB = 2000
R0_ROWS = 1280
T0 = R0_ROWS // 16
T1 = (ROWS - R0_ROWS) // 16


def _transform_body(h_ref, w_ref, t_ref, s_ref):
    t = jnp.dot(h_ref[...], w_ref[...], preferred_element_type=jnp.float32)
    t_ref[...] = t[:, : N_REL * H_DIM]
    s_ref[...] = t[:, N_REL * H_DIM:]


def _transform(h, wcat):
    d = h.shape[1]
    return pl.pallas_call(
        _transform_body,
        grid=(N // NB,),
        in_specs=[
            pl.BlockSpec((NB, d), lambda i: (i, 0)),
            pl.BlockSpec((d, (N_REL + 1) * H_DIM), lambda i: (0, 0)),
        ],
        out_specs=[
            pl.BlockSpec((NB, N_REL * H_DIM), lambda i: (i, 0)),
            pl.BlockSpec((NB, H_DIM), lambda i: (i, 0)),
        ],
        out_shape=[
            jax.ShapeDtypeStruct((N, N_REL * H_DIM), jnp.float32),
            jax.ShapeDtypeStruct((N, H_DIM), jnp.float32),
        ],
    )(h, wcat)


def _transform_fused_body(p_ref, s_ref, w_ref, t_ref, s2_ref):
    h = jnp.maximum(p_ref[0] + p_ref[1] + s_ref[...], 0.0)
    t = jnp.dot(h, w_ref[...], preferred_element_type=jnp.float32)
    t_ref[...] = t[:, : N_REL * H_DIM]
    s2_ref[...] = t[:, N_REL * H_DIM:]


def _transform_fused(partials, selfh, wcat):
    return pl.pallas_call(
        _transform_fused_body,
        grid=(N // NB,),
        in_specs=[
            pl.BlockSpec((2, NB, H_DIM), lambda i: (0, i, 0)),
            pl.BlockSpec((NB, H_DIM), lambda i: (i, 0)),
            pl.BlockSpec((H_DIM, (N_REL + 1) * H_DIM), lambda i: (0, 0)),
        ],
        out_specs=[
            pl.BlockSpec((NB, N_REL * H_DIM), lambda i: (i, 0)),
            pl.BlockSpec((NB, H_DIM), lambda i: (i, 0)),
        ],
        out_shape=[
            jax.ShapeDtypeStruct((N, N_REL * H_DIM), jnp.float32),
            jax.ShapeDtypeStruct((N, H_DIM), jnp.float32),
        ],
    )(partials, selfh, wcat)


def _classifier_body(x_ref, p_ref, s_ref, w1a_ref, w1b_ref, b1_ref, w2_ref,
                     b2_ref, o_ref):
    h = jnp.maximum(p_ref[0] + p_ref[1] + s_ref[...], 0.0)
    hid = jnp.dot(x_ref[...], w1a_ref[...], preferred_element_type=jnp.float32)
    hid = hid + jnp.dot(h, w1b_ref[...], preferred_element_type=jnp.float32)
    hid = jnp.maximum(hid + b1_ref[...], 0.0)
    o_ref[...] = jnp.dot(hid, w2_ref[...], preferred_element_type=jnp.float32) + b2_ref[...]


def _classifier(x, partials, selfh, w1a, w1b, b1, w2, b2):
    return pl.pallas_call(
        _classifier_body,
        grid=(N // NB,),
        in_specs=[
            pl.BlockSpec((NB, G_DIM), lambda i: (i, 0)),
            pl.BlockSpec((2, NB, H_DIM), lambda i: (0, i, 0)),
            pl.BlockSpec((NB, H_DIM), lambda i: (i, 0)),
            pl.BlockSpec((G_DIM, HC_DIM), lambda i: (0, 0)),
            pl.BlockSpec((H_DIM, HC_DIM), lambda i: (0, 0)),
            pl.BlockSpec((1, HC_DIM), lambda i: (0, 0)),
            pl.BlockSpec((HC_DIM, TAG_SIZE), lambda i: (0, 0)),
            pl.BlockSpec((1, TAG_SIZE), lambda i: (0, 0)),
        ],
        out_specs=pl.BlockSpec((NB, TAG_SIZE), lambda i: (i, 0)),
        out_shape=jax.ShapeDtypeStruct((N, TAG_SIZE), jnp.float32),
    )(x, partials, selfh, w1a, w1b, b1, w2, b2)


def _sc_body(table, pkt_h, en_h, zeros_h, out,
             pkt_v, en_v, gidx_v, rows_v, acc,
             sem_i, sem_g, sem_s):
    core = lax.axis_index("c")
    sid = lax.axis_index("s")
    row_base = jnp.where(core == 0, sid * T0, R0_ROWS + sid * T1)
    niter = jnp.where(core == 0, T0 // NBUF, T1 // NBUF)

    def fire_idx(ch, b):
        rb = row_base + ch * CHUNK_ROWS
        pltpu.async_copy(pkt_h.at[pl.ds(rb, CHUNK_ROWS)], pkt_v.at[b],
                         sem_i[b])
        pltpu.async_copy(en_h.at[pl.ds(rb, CHUNK_ROWS)], en_v.at[b],
                         sem_i[b])

    hl = LANES // NSPLIT

    def fire_gather(ch, b):
        rb = row_base + ch * CHUNK_ROWS
        pltpu.make_async_copy(pkt_h.at[pl.ds(rb, CHUNK_ROWS)], pkt_v.at[b],
                              sem_i[b]).wait()
        pltpu.make_async_copy(en_h.at[pl.ds(rb, CHUNK_ROWS)], en_v.at[b],
                              sem_i[b]).wait()
        for r in range(CHUNK_ROWS):
            for h in range(NSPLIT):
                for k in range(hl // 16):
                    sl = pl.ds(k * 16, 16)
                    sp = pl.ds(h * hl + k * 16, 16)
                    gidx_v[b, r * NSPLIT + h, sl] = (
                        pkt_v[b, r, 0, sp] * N_REL + pkt_v[b, r, 1, sp])
        for r in range(CHUNK_ROWS):
            for h in range(NSPLIT):
                pltpu.async_copy(table.at[gidx_v.at[b, r * NSPLIT + h]],
                                 rows_v.at[b, r, pl.ds(h * hl, hl)],
                                 sem_g[b])

    def drain_scatter(b):
        for r in range(CHUNK_ROWS):
            pltpu.make_async_copy(rows_v.at[b, r],
                                  acc.at[pkt_v.at[b, r, 2]], sem_s[b]).wait()

    def process(b):
        for r in range(CHUNK_ROWS):
            for h in range(NSPLIT):
                pltpu.make_async_copy(table.at[gidx_v.at[b, r * NSPLIT + h]],
                                      rows_v.at[b, r, pl.ds(h * hl, hl)],
                                      sem_g[b]).wait()
        for r in range(CHUNK_ROWS):
            @pl.loop(0, LANES // 16)
            def _scale(g):
                env = en_v[b, r, pl.ds(g * 16, 16)]
                for j in range(16):
                    e = env[j]
                    i = g * 16 + j
                    for k in range(H_DIM // 16):
                        sl = pl.ds(k * 16, 16)
                        rows_v[b, r, i, sl] = rows_v[b, r, i, sl] * e
        for r in range(CHUNK_ROWS):
            pltpu.async_copy(rows_v.at[b, r], acc.at[pkt_v.at[b, r, 2]],
                             sem_s[b], add=True)

    with jax.named_scope("sc_zero"):
        pltpu.sync_copy(zeros_h.at[pl.ds(sid * 624, 624)],
                        acc.at[pl.ds(sid * 624, 624)])

        @pl.when(sid == 15)
        def _():
            pltpu.sync_copy(zeros_h.at[pl.ds(9984, 16)],
                            acc.at[pl.ds(9984, 16)])

        plsc.subcore_barrier()

    @pl.when(niter > 0)
    def _():
        fire_idx(0, 0)
        fire_idx(1, 1)
        fire_idx(2, 2)
        fire_gather(0, 0)
        fire_gather(1, 1)

    with jax.named_scope("sc_main"):
        @pl.loop(0, niter)
        def _chunks(i):
            for b in range(NBUF):
                ch = i * NBUF + b
                b3 = (b + 3) % NBUF
                b2 = (b + 2) % NBUF
                if b < 2:
                    @pl.when(i > 0)
                    def _():
                        drain_scatter(b3)
                    fire_idx(ch + 3, b3)
                else:
                    drain_scatter(b3)

                    @pl.when(i < niter - 1)
                    def _():
                        fire_idx(ch + 3, b3)
                if b <= 2:
                    fire_gather(ch + 2, b2)
                else:
                    @pl.when(i < niter - 1)
                    def _():
                        fire_gather(ch + 2, b2)
                process(b)

    with jax.named_scope("sc_tail"):
        @pl.when(niter > 0)
        def _():
            drain_scatter(NBUF - 2)
            drain_scatter(NBUF - 1)

        plsc.subcore_barrier()

    with jax.named_scope("sc_dump"):
        pltpu.sync_copy(acc.at[pl.ds(sid * 624, 624)],
                        out.at[core, pl.ds(sid * 624, 624)])

        @pl.when(sid == 15)
        def _():
            pltpu.sync_copy(acc.at[pl.ds(9984, 16)],
                            out.at[core, pl.ds(9984, 16)])


def _sc_aggregate(table, pkt, en2, zeros):
    k = pl.kernel(
        _sc_body,
        out_type=jax.ShapeDtypeStruct((2, N, H_DIM), jnp.float32),
        mesh=plsc.VectorSubcoreMesh(core_axis_name="c", subcore_axis_name="s"),
        compiler_params=pltpu.CompilerParams(use_tc_tiling_on_sc=False),
        scratch_types=[
            pltpu.VMEM((NBUF, CHUNK_ROWS, 3, LANES), jnp.int32),
            pltpu.VMEM((NBUF, CHUNK_ROWS, LANES), jnp.float32),
            pltpu.VMEM((NBUF, CHUNK_ROWS * NSPLIT, LANES // NSPLIT),
                       jnp.int32),
            pltpu.VMEM((NBUF, CHUNK_ROWS, LANES, H_DIM), jnp.float32),
            pltpu.VMEM_SHARED((N, H_DIM), jnp.float32),
            [pltpu.SemaphoreType.DMA] * NBUF,
            [pltpu.SemaphoreType.DMA] * NBUF,
            [pltpu.SemaphoreType.DMA] * NBUF,
        ],
    )
    return k(table, pkt, en2, zeros)


def kernel(x, edge_index, edge_type, edge_norm, Wrel1, Wself1, Wrel, Wself,
           Wc1, bc1, Wc2, bc2):
    wcat1 = jnp.concatenate(
        [Wrel1.transpose(1, 0, 2).reshape(G_DIM, N_REL * H_DIM), Wself1], axis=1)
    wcats = [wcat1]
    for i in range(NUM_LAYERS - 1):
        wcats.append(jnp.concatenate(
            [Wrel[i].transpose(1, 0, 2).reshape(H_DIM, N_REL * H_DIM), Wself[i]],
            axis=1))

    pad = EP - E
    pad_nodes = jnp.arange(pad, dtype=jnp.int32) % N

    def _padded(a, tail):
        return jnp.concatenate([a, tail]).reshape(ROWS, LANES)

    pkt = jnp.stack([
        _padded(edge_index[0], pad_nodes),
        _padded(edge_type, jnp.zeros((pad,), jnp.int32)),
        _padded(edge_index[1], pad_nodes),
    ], axis=1)
    en2 = _padded(edge_norm, jnp.zeros((pad,), jnp.float32))
    zeros = jnp.zeros((N, H_DIM), jnp.float32)

    t_rel, selfh = _transform(x, wcats[0])
    partials = _sc_aggregate(t_rel.reshape(N * N_REL, H_DIM), pkt, en2, zeros)
    for l in range(1, NUM_LAYERS):
        t_rel, selfh = _transform_fused(partials, selfh, wcats[l])
        partials = _sc_aggregate(t_rel.reshape(N * N_REL, H_DIM), pkt, en2, zeros)

    logits = _classifier(x, partials, selfh, Wc1[:G_DIM], Wc1[G_DIM:],
                         bc1.reshape(1, HC_DIM), Wc2, bc2.reshape(1, TAG_SIZE))
    return logits
